# baseline (speedup 1.0000x reference)
import jax
import jax.numpy as jnp
from jax.experimental import pallas as pl
from jax.experimental.pallas import tpu as pltpu

E = 8
TOP_K = 2
DIM = 768
DFF = DIM * 4
T = 2048
NT = T * TOP_K
BT = 128
NB = NT // BT


NC = 4
CH = T // NC


def _moe_body(x_hbm, W1_ref, b1_ref, W2_ref, b2_ref, Wg_ref, bg_ref,
              out_ref, xc_ref, xb_ref, p1_ref, p2_ref, a1_ref, a2_ref,
              w1_ref, w2_ref, offs_ref, dma_sem):
    e = pl.program_id(0)
    b = pl.program_id(1)

    @pl.when((e == 0) & (b == 0))
    def _router():
        logit_chunks = []
        for c in range(NC):
            cp = pltpu.make_async_copy(
                x_hbm.at[pl.ds(c * CH, CH), :], xc_ref, dma_sem)
            cp.start()
            cp.wait()
            xc = xc_ref[...]
            xb_ref[pl.ds(c * CH, CH), :] = xc.astype(jnp.bfloat16)
            logit_chunks.append(
                jnp.dot(xc, Wg_ref[...],
                        preferred_element_type=jnp.float32) + bg_ref[...])
        logits = jnp.concatenate(logit_chunks, axis=0)
        m1 = jnp.full((T, 1), -jnp.inf, jnp.float32)
        a1 = jnp.zeros((T, 1), jnp.int32)
        for k in range(E):
            lk = logits[:, k:k + 1]
            better = lk > m1
            a1 = jnp.where(better, k, a1)
            m1 = jnp.where(better, lk, m1)
        m2 = jnp.full((T, 1), -jnp.inf, jnp.float32)
        a2 = jnp.zeros((T, 1), jnp.int32)
        for k in range(E):
            lk = logits[:, k:k + 1]
            better = (lk > m2) & (a1 != k)
            a2 = jnp.where(better, k, a2)
            m2 = jnp.where(better, lk, m2)
        ex2 = jnp.exp(m2 - m1)
        denom = 1.0 + ex2
        w1_ref[...] = 1.0 / denom
        w2_ref[...] = ex2 / denom
        a1_ref[...] = a1
        a2_ref[...] = a2

        iota_e = jax.lax.broadcasted_iota(jnp.int32, (T, E), 1)
        M = ((iota_e == a1) | (iota_e == a2)).astype(jnp.bfloat16)
        rank_chunks = []
        for rb in range(T // BT):
            row = jax.lax.broadcasted_iota(jnp.int32, (BT, T), 0) + rb * BT
            col = jax.lax.broadcasted_iota(jnp.int32, (BT, T), 1)
            Lc = (col < row).astype(jnp.bfloat16)
            rank_chunks.append(jnp.dot(Lc, M,
                                       preferred_element_type=jnp.float32))
        ranks = jnp.concatenate(rank_chunks, axis=0)

        offs = 0
        offs_ref[0] = 0
        off_list = []
        for k in range(E):
            off_list.append(offs)
            cnt = jnp.sum(M[:, k:k + 1].astype(jnp.float32)).astype(jnp.int32)
            offs = offs + cnt
            offs_ref[k + 1] = offs

        ri = ranks.astype(jnp.int32)
        p1 = jnp.zeros((T, 1), jnp.int32)
        p2 = jnp.zeros((T, 1), jnp.int32)
        for k in range(E):
            rk = ri[:, k:k + 1]
            p1 = jnp.where(a1 == k, off_list[k] + rk, p1)
            p2 = jnp.where(a2 == k, off_list[k] + rk, p2)
        p1_ref[...] = p1
        p2_ref[...] = p2

    @pl.when((e == 0) & (b == 0))
    def _zero():
        out_ref[...] = jnp.zeros_like(out_ref)

    base = b * BT
    active = (offs_ref[e] < base + BT) & (offs_ref[e + 1] > base)

    @pl.when(active)
    def _ffn_block():
        lane = jax.lax.broadcasted_iota(jnp.int32, (T, BT), 1) + base
        hit1 = (p1_ref[...] == lane) & (a1_ref[...] == e)
        hit2 = (p2_ref[...] == lane) & (a2_ref[...] == e)
        PT = (hit1 | hit2).astype(jnp.bfloat16)
        dn = (((0,), (0,)), ((), ()))
        xs = jax.lax.dot_general(PT, xb_ref[...], dn,
                                 preferred_element_type=jnp.float32)
        xs = xs.astype(jnp.bfloat16)
        wsel = (jnp.where(hit1, w1_ref[...], 0.0)
                + jnp.where(hit2, w2_ref[...], 0.0)).astype(jnp.bfloat16)
        wrow = jax.lax.dot_general(wsel, jnp.ones((T, 1), jnp.bfloat16), dn,
                                   preferred_element_type=jnp.float32)
        h = jnp.dot(xs, W1_ref[0].astype(jnp.bfloat16),
                    preferred_element_type=jnp.float32) + b1_ref[0]
        h = (h * 0.5 * (1.0 + jax.lax.erf(h * 0.7071067811865476)))
        h = h.astype(jnp.bfloat16)
        ys = jnp.dot(h, W2_ref[0].astype(jnp.bfloat16),
                     preferred_element_type=jnp.float32) + b2_ref[0]
        ys = (ys * wrow).astype(jnp.bfloat16)
        upd = jnp.dot(PT, ys, preferred_element_type=jnp.float32)
        out_ref[...] = (out_ref[...].astype(jnp.float32)
                        + upd).astype(jnp.bfloat16)


def kernel(x, W1, b1, W2, b2, Wg, bg):
    B, S, _ = x.shape
    x2 = x.reshape(S, DIM)
    bg2 = bg.reshape(1, E)
    b1r = b1.reshape(E, 1, DFF)
    b2r = b2.reshape(E, 1, DIM)

    out = pl.pallas_call(
        _moe_body,
        grid=(E, NB),
        in_specs=[
            pl.BlockSpec(memory_space=pl.ANY),
            pl.BlockSpec((1, DIM, DFF), lambda e, b: (e, 0, 0)),
            pl.BlockSpec((1, 1, DFF), lambda e, b: (e, 0, 0)),
            pl.BlockSpec((1, DFF, DIM), lambda e, b: (e, 0, 0)),
            pl.BlockSpec((1, 1, DIM), lambda e, b: (e, 0, 0)),
            pl.BlockSpec((DIM, E), lambda e, b: (0, 0)),
            pl.BlockSpec((1, E), lambda e, b: (0, 0)),
        ],
        out_specs=pl.BlockSpec((T, DIM), lambda e, b: (0, 0)),
        out_shape=jax.ShapeDtypeStruct((T, DIM), jnp.bfloat16),
        scratch_shapes=[
            pltpu.VMEM((CH, DIM), jnp.float32),
            pltpu.VMEM((T, DIM), jnp.bfloat16),
            pltpu.VMEM((T, 1), jnp.int32),
            pltpu.VMEM((T, 1), jnp.int32),
            pltpu.VMEM((T, 1), jnp.int32),
            pltpu.VMEM((T, 1), jnp.int32),
            pltpu.VMEM((T, 1), jnp.float32),
            pltpu.VMEM((T, 1), jnp.float32),
            pltpu.SMEM((16,), jnp.int32),
            pltpu.SemaphoreType.DMA,
        ],
        compiler_params=pltpu.CompilerParams(
            dimension_semantics=("arbitrary", "arbitrary"),
            vmem_limit_bytes=64 * 1024 * 1024,
        ),
    )(x2, W1, b1r, W2, b2r, Wg, bg2)
    return out.astype(jnp.float32).reshape(B, S, DIM)


# speedup vs baseline: 1.4053x; 1.4053x over previous
import jax
import jax.numpy as jnp
from jax.experimental import pallas as pl
from jax.experimental.pallas import tpu as pltpu

E = 8
TOP_K = 2
DIM = 768
DFF = DIM * 4
T = 2048
NT = T * TOP_K
BT = 256
NTP = NT + E * BT
NBP = NTP // BT
BTC = 256
SQRT1_2 = 0.7071067811865476


def _router_body(x_ref, Wg_ref, bg_ref,
                 xb_ref, a1_ref, a2_ref, w1_ref, w2_ref,
                 ranks_ref, p1_ref, p2_ref, be_ref):
    x = x_ref[...]
    xb_ref[...] = x.astype(jnp.bfloat16)
    logits = jnp.dot(x, Wg_ref[...],
                     preferred_element_type=jnp.float32) + bg_ref[...]
    m1 = jnp.full((T, 1), -jnp.inf, jnp.float32)
    a1 = jnp.zeros((T, 1), jnp.int32)
    for k in range(E):
        lk = logits[:, k:k + 1]
        better = lk > m1
        a1 = jnp.where(better, k, a1)
        m1 = jnp.where(better, lk, m1)
    m2 = jnp.full((T, 1), -jnp.inf, jnp.float32)
    a2 = jnp.zeros((T, 1), jnp.int32)
    for k in range(E):
        lk = logits[:, k:k + 1]
        better = (lk > m2) & (a1 != k)
        a2 = jnp.where(better, k, a2)
        m2 = jnp.where(better, lk, m2)
    ex2 = jnp.exp(m2 - m1)
    denom = 1.0 + ex2
    w1_ref[...] = 1.0 / denom
    w2_ref[...] = ex2 / denom
    a1_ref[...] = a1
    a2_ref[...] = a2

    iota_e = jax.lax.broadcasted_iota(jnp.int32, (T, E), 1)
    M = ((iota_e == a1) | (iota_e == a2)).astype(jnp.bfloat16)
    rank_chunks = []
    for rb in range(T // 256):
        row = jax.lax.broadcasted_iota(jnp.int32, (256, T), 0) + rb * 256
        col = jax.lax.broadcasted_iota(jnp.int32, (256, T), 1)
        Lc = (col < row).astype(jnp.bfloat16)
        rank_chunks.append(jnp.dot(Lc, M,
                                   preferred_element_type=jnp.float32))
    ranks = jnp.concatenate(rank_chunks, axis=0).astype(jnp.int32)
    ranks_ref[...] = ranks

    offs = 0
    off_list = []
    for k in range(E):
        off_list.append(offs)
        cnt = jnp.sum(M[:, k:k + 1].astype(jnp.float32)).astype(jnp.int32)
        pcnt = ((cnt + BT - 1) // BT) * BT
        offs = offs + pcnt
    blk_base = jax.lax.broadcasted_iota(jnp.int32, (1, NBP), 1) * BT
    be = jnp.zeros((1, NBP), jnp.int32)
    for k in range(1, E):
        be = be + (blk_base >= off_list[k]).astype(jnp.int32)
    off_iota = jax.lax.broadcasted_iota(jnp.int32, (1, E), 1)
    off_row = jnp.zeros((1, E), jnp.int32)
    for k in range(E):
        off_row = jnp.where(off_iota == k, off_list[k], off_row)
    be_ref[...] = jnp.concatenate([be, off_row], axis=1)

    p1 = jnp.zeros((T, 1), jnp.int32)
    p2 = jnp.zeros((T, 1), jnp.int32)
    for k in range(E):
        rk = ranks[:, k:k + 1]
        p1 = jnp.where(a1 == k, off_list[k] + rk, p1)
        p2 = jnp.where(a2 == k, off_list[k] + rk, p2)
    p1_ref[...] = p1
    p2_ref[...] = p2


def _ffn_body(meta_ref, xb_ref, a1_ref, a2_ref, w1_ref, w2_ref, ranks_ref,
              W1_ref, b1_ref, W2_ref, b2_ref, ys_ref):
    b = pl.program_id(0)
    eb = meta_ref[b]
    base = b * BT
    lbase = base - meta_ref[NBP + eb]

    iota_e = jax.lax.broadcasted_iota(jnp.int32, (T, E), 1)
    eq1 = a1_ref[...] == eb
    eq2 = a2_ref[...] == eb
    m_col = eq1 | eq2
    rank_col = jnp.sum(jnp.where(iota_e == eb, ranks_ref[...], 0),
                       axis=1, keepdims=True)
    lane = jax.lax.broadcasted_iota(jnp.int32, (T, BT), 1)
    hit = (rank_col - lbase == lane) & m_col
    PT = hit.astype(jnp.bfloat16)
    wcol = jnp.where(eq1, w1_ref[...], jnp.where(eq2, w2_ref[...], 0.0))

    dn = (((0,), (0,)), ((), ()))
    xs = jax.lax.dot_general(PT, xb_ref[...], dn,
                             preferred_element_type=jnp.float32)
    xs = xs.astype(jnp.bfloat16)
    wrow = jax.lax.dot_general(
        jnp.where(hit, wcol, 0.0).astype(jnp.bfloat16),
        jnp.ones((T, 1), jnp.bfloat16), dn,
        preferred_element_type=jnp.float32)

    h = jnp.dot(xs, W1_ref[0].astype(jnp.bfloat16),
                preferred_element_type=jnp.float32) + b1_ref[0]
    h = (h * 0.5 * (1.0 + jax.lax.erf(h * SQRT1_2))).astype(jnp.bfloat16)
    ys = jnp.dot(h, W2_ref[0].astype(jnp.bfloat16),
                 preferred_element_type=jnp.float32) + b2_ref[0]
    ys_ref[...] = (ys * wrow).astype(jnp.bfloat16)


def _combine_body(p1_ref, p2_ref, ys_ref, out_ref):
    lane = jax.lax.broadcasted_iota(jnp.int32, (BTC, NTP), 1)
    P = ((p1_ref[...] == lane).astype(jnp.bfloat16)
         + (p2_ref[...] == lane).astype(jnp.bfloat16))
    out_ref[...] = jnp.dot(P, ys_ref[...],
                           preferred_element_type=jnp.float32)


def kernel(x, W1, b1, W2, b2, Wg, bg):
    B, S, _ = x.shape
    x2 = x.reshape(S, DIM)
    bg2 = bg.reshape(1, E)
    b1r = b1.reshape(E, 1, DFF)
    b2r = b2.reshape(E, 1, DIM)

    xb, a1, a2, w1, w2, ranks, p1, p2, meta = pl.pallas_call(
        _router_body,
        grid=(1,),
        in_specs=[
            pl.BlockSpec((T, DIM), lambda i: (0, 0)),
            pl.BlockSpec((DIM, E), lambda i: (0, 0)),
            pl.BlockSpec((1, E), lambda i: (0, 0)),
        ],
        out_specs=[
            pl.BlockSpec((T, DIM), lambda i: (0, 0)),
            pl.BlockSpec((T, 1), lambda i: (0, 0)),
            pl.BlockSpec((T, 1), lambda i: (0, 0)),
            pl.BlockSpec((T, 1), lambda i: (0, 0)),
            pl.BlockSpec((T, 1), lambda i: (0, 0)),
            pl.BlockSpec((T, E), lambda i: (0, 0)),
            pl.BlockSpec((T, 1), lambda i: (0, 0)),
            pl.BlockSpec((T, 1), lambda i: (0, 0)),
            pl.BlockSpec((1, NBP + E), lambda i: (0, 0)),
        ],
        out_shape=[
            jax.ShapeDtypeStruct((T, DIM), jnp.bfloat16),
            jax.ShapeDtypeStruct((T, 1), jnp.int32),
            jax.ShapeDtypeStruct((T, 1), jnp.int32),
            jax.ShapeDtypeStruct((T, 1), jnp.float32),
            jax.ShapeDtypeStruct((T, 1), jnp.float32),
            jax.ShapeDtypeStruct((T, E), jnp.int32),
            jax.ShapeDtypeStruct((T, 1), jnp.int32),
            jax.ShapeDtypeStruct((T, 1), jnp.int32),
            jax.ShapeDtypeStruct((1, NBP + E), jnp.int32),
        ],
    )(x2, Wg, bg2)

    grid_spec = pltpu.PrefetchScalarGridSpec(
        num_scalar_prefetch=1,
        grid=(NBP,),
        in_specs=[
            pl.BlockSpec((T, DIM), lambda b, m: (0, 0)),
            pl.BlockSpec((T, 1), lambda b, m: (0, 0)),
            pl.BlockSpec((T, 1), lambda b, m: (0, 0)),
            pl.BlockSpec((T, 1), lambda b, m: (0, 0)),
            pl.BlockSpec((T, 1), lambda b, m: (0, 0)),
            pl.BlockSpec((T, E), lambda b, m: (0, 0)),
            pl.BlockSpec((1, DIM, DFF), lambda b, m: (m[b], 0, 0)),
            pl.BlockSpec((1, 1, DFF), lambda b, m: (m[b], 0, 0)),
            pl.BlockSpec((1, DFF, DIM), lambda b, m: (m[b], 0, 0)),
            pl.BlockSpec((1, 1, DIM), lambda b, m: (m[b], 0, 0)),
        ],
        out_specs=pl.BlockSpec((BT, DIM), lambda b, m: (b, 0)),
    )
    ys = pl.pallas_call(
        _ffn_body,
        grid_spec=grid_spec,
        out_shape=jax.ShapeDtypeStruct((NTP, DIM), jnp.bfloat16),
    )(meta.reshape(-1), xb, a1, a2, w1, w2, ranks, W1, b1r, W2, b2r)

    out = pl.pallas_call(
        _combine_body,
        grid=(T // BTC,),
        in_specs=[
            pl.BlockSpec((BTC, 1), lambda b: (b, 0)),
            pl.BlockSpec((BTC, 1), lambda b: (b, 0)),
            pl.BlockSpec((NTP, DIM), lambda b: (0, 0)),
        ],
        out_specs=pl.BlockSpec((BTC, DIM), lambda b: (b, 0)),
        out_shape=jax.ShapeDtypeStruct((T, DIM), jnp.float32),
    )(p1, p2, ys)
    return out.reshape(B, S, DIM)


# speedup vs baseline: 1.7786x; 1.2656x over previous
import jax
import jax.numpy as jnp
from jax.experimental import pallas as pl
from jax.experimental.pallas import tpu as pltpu

E = 8
TOP_K = 2
DIM = 768
DFF = DIM * 4
T = 2048
NT = T * TOP_K
BT = 256
NTP = NT + E * BT
NBP = NTP // BT
MW = NBP + 8
BTC = 256
SQRT1_2 = 0.7071067811865476


def _router_body(x_ref, Wg_ref, bg_ref,
                 xb_ref, w1_ref, w2_ref, p1_ref, p2_ref, be_ref):
    x = x_ref[...]
    xb_ref[...] = x.astype(jnp.bfloat16)
    logits = jnp.dot(x, Wg_ref[...],
                     preferred_element_type=jnp.float32) + bg_ref[...]
    iota_e = jax.lax.broadcasted_iota(jnp.int32, (T, E), 1)
    m1 = jnp.max(logits, axis=1, keepdims=True)
    a1 = jnp.min(jnp.where(logits == m1, iota_e, E), axis=1, keepdims=True)
    oh1 = iota_e == a1
    l2 = jnp.where(oh1, -jnp.inf, logits)
    m2 = jnp.max(l2, axis=1, keepdims=True)
    a2 = jnp.min(jnp.where(l2 == m2, iota_e, E), axis=1, keepdims=True)
    oh2 = iota_e == a2
    ex2 = jnp.exp(m2 - m1)
    denom = 1.0 + ex2
    w1_ref[...] = 1.0 / denom
    w2_ref[...] = ex2 / denom

    M = (oh1 | oh2).astype(jnp.bfloat16)
    rank_chunks = []
    for rb in range(T // 256):
        row = jax.lax.broadcasted_iota(jnp.int32, (256, T), 0) + rb * 256
        col = jax.lax.broadcasted_iota(jnp.int32, (256, T), 1)
        Lc = (col < row).astype(jnp.bfloat16)
        rank_chunks.append(jnp.dot(Lc, M,
                                   preferred_element_type=jnp.float32))
    ranks = jnp.concatenate(rank_chunks, axis=0)

    last = ranks[T - 1:T, :] + M[T - 1:T, :].astype(jnp.float32)
    offs = 0
    off_list = []
    for k in range(E):
        off_list.append(offs)
        cnt = jnp.sum(last[:, k:k + 1]).astype(jnp.int32)
        offs = offs + ((cnt + BT - 1) // BT) * BT
    blk_base = jax.lax.broadcasted_iota(jnp.int32, (1, MW), 1) * BT
    be = jnp.zeros((1, MW), jnp.int32)
    for k in range(1, E):
        be = be + (blk_base >= off_list[k]).astype(jnp.int32)
    lane_m = jax.lax.broadcasted_iota(jnp.int32, (1, MW), 1)
    be = jnp.where(lane_m == NBP, offs, jnp.where(lane_m > NBP, 0, be))
    be_ref[...] = be

    off_row = jnp.zeros((1, E), jnp.int32)
    off_iota = jax.lax.broadcasted_iota(jnp.int32, (1, E), 1)
    for k in range(E):
        off_row = jnp.where(off_iota == k, off_list[k], off_row)
    pos = ranks.astype(jnp.int32) + off_row
    p1_ref[...] = jnp.sum(jnp.where(oh1, pos, 0), axis=1, keepdims=True)
    p2_ref[...] = jnp.sum(jnp.where(oh2, pos, 0), axis=1, keepdims=True)


def _ffn_body(meta_ref, xb_ref, p1_ref, p2_ref, w1_ref, w2_ref,
              W1_ref, b1_ref, W2_ref, b2_ref, ys_ref):
    b = pl.program_id(0)
    base = b * BT

    @pl.when(base < meta_ref[NBP])
    def _compute():
        lane = jax.lax.broadcasted_iota(jnp.int32, (T, BT), 1) + base
        hit1 = p1_ref[...] == lane
        hit2 = p2_ref[...] == lane
        PT = (hit1 | hit2).astype(jnp.bfloat16)
        wsel = (jnp.where(hit1, w1_ref[...], 0.0)
                + jnp.where(hit2, w2_ref[...], 0.0)).astype(jnp.bfloat16)

        dn = (((0,), (0,)), ((), ()))
        xs = jax.lax.dot_general(PT, xb_ref[...], dn,
                                 preferred_element_type=jnp.float32)
        xs = xs.astype(jnp.bfloat16)
        wrow = jax.lax.dot_general(wsel, jnp.ones((T, 1), jnp.bfloat16), dn,
                                   preferred_element_type=jnp.float32)

        h = jnp.dot(xs, W1_ref[0].astype(jnp.bfloat16),
                    preferred_element_type=jnp.float32) + b1_ref[0]
        h = (h * 0.5 * (1.0 + jax.lax.erf(h * SQRT1_2))).astype(jnp.bfloat16)
        ys = jnp.dot(h, W2_ref[0].astype(jnp.bfloat16),
                     preferred_element_type=jnp.float32) + b2_ref[0]
        ys_ref[...] = (ys * wrow).astype(jnp.bfloat16)

    @pl.when(base >= meta_ref[NBP])
    def _pad():
        ys_ref[...] = jnp.zeros_like(ys_ref)


def _combine_body(p1_ref, p2_ref, ys_ref, out_ref):
    lane = jax.lax.broadcasted_iota(jnp.int32, (BTC, NTP), 1)
    P = ((p1_ref[...] == lane) | (p2_ref[...] == lane)).astype(jnp.bfloat16)
    out_ref[...] = jnp.dot(P, ys_ref[...],
                           preferred_element_type=jnp.float32)


def kernel(x, W1, b1, W2, b2, Wg, bg):
    B, S, _ = x.shape
    x2 = x.reshape(S, DIM)
    bg2 = bg.reshape(1, E)
    b1r = b1.reshape(E, 1, DFF)
    b2r = b2.reshape(E, 1, DIM)

    xb, w1, w2, p1, p2, meta = pl.pallas_call(
        _router_body,
        grid=(1,),
        in_specs=[
            pl.BlockSpec((T, DIM), lambda i: (0, 0)),
            pl.BlockSpec((DIM, E), lambda i: (0, 0)),
            pl.BlockSpec((1, E), lambda i: (0, 0)),
        ],
        out_specs=[
            pl.BlockSpec((T, DIM), lambda i: (0, 0)),
            pl.BlockSpec((T, 1), lambda i: (0, 0)),
            pl.BlockSpec((T, 1), lambda i: (0, 0)),
            pl.BlockSpec((T, 1), lambda i: (0, 0)),
            pl.BlockSpec((T, 1), lambda i: (0, 0)),
            pl.BlockSpec((1, MW), lambda i: (0, 0)),
        ],
        out_shape=[
            jax.ShapeDtypeStruct((T, DIM), jnp.bfloat16),
            jax.ShapeDtypeStruct((T, 1), jnp.float32),
            jax.ShapeDtypeStruct((T, 1), jnp.float32),
            jax.ShapeDtypeStruct((T, 1), jnp.int32),
            jax.ShapeDtypeStruct((T, 1), jnp.int32),
            jax.ShapeDtypeStruct((1, MW), jnp.int32),
        ],
    )(x2, Wg, bg2)

    grid_spec = pltpu.PrefetchScalarGridSpec(
        num_scalar_prefetch=1,
        grid=(NBP,),
        in_specs=[
            pl.BlockSpec((T, DIM), lambda b, m: (0, 0)),
            pl.BlockSpec((T, 1), lambda b, m: (0, 0)),
            pl.BlockSpec((T, 1), lambda b, m: (0, 0)),
            pl.BlockSpec((T, 1), lambda b, m: (0, 0)),
            pl.BlockSpec((T, 1), lambda b, m: (0, 0)),
            pl.BlockSpec((1, DIM, DFF), lambda b, m: (m[b], 0, 0)),
            pl.BlockSpec((1, 1, DFF), lambda b, m: (m[b], 0, 0)),
            pl.BlockSpec((1, DFF, DIM), lambda b, m: (m[b], 0, 0)),
            pl.BlockSpec((1, 1, DIM), lambda b, m: (m[b], 0, 0)),
        ],
        out_specs=pl.BlockSpec((BT, DIM), lambda b, m: (b, 0)),
    )
    ys = pl.pallas_call(
        _ffn_body,
        grid_spec=grid_spec,
        out_shape=jax.ShapeDtypeStruct((NTP, DIM), jnp.bfloat16),
    )(meta.reshape(-1), xb, p1, p2, w1, w2, W1, b1r, W2, b2r)

    out = pl.pallas_call(
        _combine_body,
        grid=(T // BTC,),
        in_specs=[
            pl.BlockSpec((BTC, 1), lambda b: (b, 0)),
            pl.BlockSpec((BTC, 1), lambda b: (b, 0)),
            pl.BlockSpec((NTP, DIM), lambda b: (0, 0)),
        ],
        out_specs=pl.BlockSpec((BTC, DIM), lambda b: (b, 0)),
        out_shape=jax.ShapeDtypeStruct((T, DIM), jnp.float32),
    )(p1, p2, ys)
    return out.reshape(B, S, DIM)
